# baseline (speedup 1.0000x reference)
import functools

import jax
import jax.numpy as jnp
from jax import lax
from jax.experimental import pallas as pl
from jax.experimental.pallas import tpu as pltpu
from jax.experimental.pallas import tpu_sc as plsc

_R = 32


def _make_sc_add(B, S, D):
    info = plsc.get_sparse_core_info()
    NC, NS = info.num_cores, info.num_subcores
    NW = NC * NS
    rows_per_w = S // NW
    CH = rows_per_w // _R
    CW = _R * D

    mesh = plsc.VectorSubcoreMesh(core_axis_name="c", subcore_axis_name="s")

    @functools.partial(
        pl.kernel,
        mesh=mesh,
        out_type=jax.ShapeDtypeStruct((B * S * D,), jnp.float32),
        scratch_types=[
            pltpu.VMEM((CW,), jnp.float32),
            pltpu.VMEM((CW,), jnp.float32),
        ],
    )
    def sc_add(tok_hbm, pos_hbm, out_hbm, pos_v, tok_v):
        wid = lax.axis_index("s") * NC + lax.axis_index("c")
        s0 = wid * rows_per_w
        for c in range(CH):
            off_p = (s0 + c * _R) * D
            pltpu.sync_copy(pos_hbm.at[pl.ds(off_p, CW)], pos_v)
            for b in range(B):
                off_t = b * S * D + off_p
                pltpu.sync_copy(tok_hbm.at[pl.ds(off_t, CW)], tok_v)

                def add16(i, _):
                    sl = pl.ds(i * 16, 16)
                    tok_v[sl] = tok_v[sl] + pos_v[sl]
                    return 0

                lax.fori_loop(0, CW // 16, add16, 0)
                pltpu.sync_copy(tok_v, out_hbm.at[pl.ds(off_t, CW)])

    return sc_add


def kernel(encoded_tokens, pos_table):
    B, S, D = encoded_tokens.shape
    sc_add = _make_sc_add(B, S, D)
    out = sc_add(encoded_tokens.reshape(B * S * D), pos_table.reshape(S * D))
    return out.reshape(B, S, D)


# speedup vs baseline: 1.7096x; 1.7096x over previous
import functools

import jax
import jax.numpy as jnp
from jax import lax
from jax.experimental import pallas as pl
from jax.experimental.pallas import tpu as pltpu
from jax.experimental.pallas import tpu_sc as plsc

_R = 32


def _make_sc_add(B, S, D):
    info = plsc.get_sparse_core_info()
    NC, NS = info.num_cores, info.num_subcores
    NW = NC * NS
    rows_per_w = S // NW
    CH = rows_per_w // _R
    CW = _R * D
    NIT = CH * B

    mesh = plsc.VectorSubcoreMesh(core_axis_name="c", subcore_axis_name="s")

    @functools.partial(
        pl.kernel,
        mesh=mesh,
        out_type=jax.ShapeDtypeStruct((B * S * D,), jnp.float32),
        scratch_types=[
            pltpu.VMEM((CW,), jnp.float32),
            pltpu.VMEM((CW,), jnp.float32),
            pltpu.VMEM((CW,), jnp.float32),
            pltpu.VMEM((CW,), jnp.float32),
            pltpu.SemaphoreType.DMA,
            pltpu.SemaphoreType.DMA,
            pltpu.SemaphoreType.DMA,
            pltpu.SemaphoreType.DMA,
            pltpu.SemaphoreType.DMA,
            pltpu.SemaphoreType.DMA,
        ],
    )
    def sc_add(tok_hbm, pos_hbm, out_hbm, pos_a, pos_b, tok_a, tok_b,
               psem_a, psem_b, isem_a, isem_b, osem_a, osem_b):
        wid = lax.axis_index("s") * NC + lax.axis_index("c")
        s0 = wid * rows_per_w
        pos_bufs, psems = (pos_a, pos_b), (psem_a, psem_b)
        tok_bufs, isems = (tok_a, tok_b), (isem_a, isem_b)
        osems = (osem_a, osem_b)

        def pos_off(c):
            return (s0 + c * _R) * D

        def tok_off(k):
            c, b = divmod(k, B)
            return b * S * D + pos_off(c)

        pos_desc = [None, None]
        tok_desc = [None, None]
        out_desc = [None, None]
        pos_desc[0] = pltpu.async_copy(
            pos_hbm.at[pl.ds(pos_off(0), CW)], pos_bufs[0], psems[0])
        tok_desc[0] = pltpu.async_copy(
            tok_hbm.at[pl.ds(tok_off(0), CW)], tok_bufs[0], isems[0])

        for k in range(NIT):
            c, b = divmod(k, B)
            bi = k % 2
            pi = c % 2
            if b == 0:
                pos_desc[pi].wait()
                if c + 1 < CH:
                    ni = (c + 1) % 2
                    pos_desc[ni] = pltpu.async_copy(
                        pos_hbm.at[pl.ds(pos_off(c + 1), CW)], pos_bufs[ni], psems[ni])
            tok_desc[bi].wait()
            if k + 1 < NIT:
                ni = (k + 1) % 2
                if out_desc[ni] is not None:
                    out_desc[ni].wait()
                tok_desc[ni] = pltpu.async_copy(
                    tok_hbm.at[pl.ds(tok_off(k + 1), CW)], tok_bufs[ni], isems[ni])

            tok_v, pos_v = tok_bufs[bi], pos_bufs[pi]

            @plsc.parallel_loop(0, CW, step=16, unroll=8)
            def add16(i):
                sl = pl.ds(i, 16)
                tok_v[sl] = tok_v[sl] + pos_v[sl]

            out_desc[bi] = pltpu.async_copy(
                tok_v, out_hbm.at[pl.ds(tok_off(k), CW)], osems[bi])

        for d in out_desc:
            if d is not None:
                d.wait()

    return sc_add


def kernel(encoded_tokens, pos_table):
    B, S, D = encoded_tokens.shape
    sc_add = _make_sc_add(B, S, D)
    out = sc_add(encoded_tokens.reshape(B * S * D), pos_table.reshape(S * D))
    return out.reshape(B, S, D)


# speedup vs baseline: 1.7148x; 1.0030x over previous
import functools

import jax
import jax.numpy as jnp
from jax import lax
from jax.experimental import pallas as pl
from jax.experimental.pallas import tpu as pltpu
from jax.experimental.pallas import tpu_sc as plsc

_R = 32


def _make_sc_add(B, S, D):
    info = plsc.get_sparse_core_info()
    NC, NS = info.num_cores, info.num_subcores
    NW = NC * NS
    rows_per_w = S // NW
    CH = rows_per_w // _R
    CW = _R * D
    NIT = CH * B

    mesh = plsc.VectorSubcoreMesh(core_axis_name="c", subcore_axis_name="s")

    @functools.partial(
        pl.kernel,
        mesh=mesh,
        out_type=jax.ShapeDtypeStruct((B * S * D,), jnp.float32),
        scratch_types=[
            pltpu.VMEM((CW,), jnp.float32),
            pltpu.VMEM((CW,), jnp.float32),
            pltpu.VMEM((CW,), jnp.float32),
            pltpu.VMEM((CW,), jnp.float32),
            pltpu.SemaphoreType.DMA,
            pltpu.SemaphoreType.DMA,
            pltpu.SemaphoreType.DMA,
            pltpu.SemaphoreType.DMA,
            pltpu.SemaphoreType.DMA,
            pltpu.SemaphoreType.DMA,
        ],
    )
    def sc_add(tok_hbm, pos_hbm, out_hbm, pos_a, pos_b, tok_a, tok_b,
               psem_a, psem_b, isem_a, isem_b, osem_a, osem_b):
        wid = lax.axis_index("s") * NC + lax.axis_index("c")
        s0 = wid * rows_per_w
        pos_bufs, psems = (pos_a, pos_b), (psem_a, psem_b)
        tok_bufs, isems = (tok_a, tok_b), (isem_a, isem_b)
        osems = (osem_a, osem_b)

        def pos_off(c):
            return (s0 + c * _R) * D

        def tok_off(k):
            c, b = divmod(k, B)
            return b * S * D + pos_off(c)

        pos_desc = [None, None]
        tok_desc = [None, None]
        out_desc = [None, None]
        pos_desc[0] = pltpu.async_copy(
            pos_hbm.at[pl.ds(pos_off(0), CW)], pos_bufs[0], psems[0])
        tok_desc[0] = pltpu.async_copy(
            tok_hbm.at[pl.ds(tok_off(0), CW)], tok_bufs[0], isems[0])

        for k in range(NIT):
            c, b = divmod(k, B)
            bi = k % 2
            pi = c % 2
            if b == 0:
                pos_desc[pi].wait()
                if c + 1 < CH:
                    ni = (c + 1) % 2
                    pos_desc[ni] = pltpu.async_copy(
                        pos_hbm.at[pl.ds(pos_off(c + 1), CW)], pos_bufs[ni], psems[ni])
            tok_desc[bi].wait()
            if k + 1 < NIT:
                ni = (k + 1) % 2
                if out_desc[ni] is not None:
                    out_desc[ni].wait()
                tok_desc[ni] = pltpu.async_copy(
                    tok_hbm.at[pl.ds(tok_off(k + 1), CW)], tok_bufs[ni], isems[ni])

            tok_v, pos_v = tok_bufs[bi], pos_bufs[pi]

            @plsc.parallel_loop(0, CW, step=16, unroll=8)
            def add16(i):
                sl = pl.ds(i, 16)
                plsc.addupdate(tok_v.at[sl], pos_v[sl])

            out_desc[bi] = pltpu.async_copy(
                tok_v, out_hbm.at[pl.ds(tok_off(k), CW)], osems[bi])

        for d in out_desc:
            if d is not None:
                d.wait()

    return sc_add


def kernel(encoded_tokens, pos_table):
    B, S, D = encoded_tokens.shape
    sc_add = _make_sc_add(B, S, D)
    out = sc_add(encoded_tokens.reshape(B * S * D), pos_table.reshape(S * D))
    return out.reshape(B, S, D)


# speedup vs baseline: 4.6703x; 2.7235x over previous
import functools

import jax
import jax.numpy as jnp
from jax import lax
from jax.experimental import pallas as pl
from jax.experimental.pallas import tpu as pltpu
from jax.experimental.pallas import tpu_sc as plsc

_R = 32


def _make_sc_add(B, S, D):
    info = plsc.get_sparse_core_info()
    NC, NS = info.num_cores, info.num_subcores
    NW = NC * NS
    rows_per_w = S // NW
    CH = rows_per_w // _R
    NIT = CH * B

    mesh = plsc.VectorSubcoreMesh(core_axis_name="c", subcore_axis_name="s")

    @functools.partial(
        pl.kernel,
        mesh=mesh,
        out_type=jax.ShapeDtypeStruct((B * S, D), jnp.float32),
        scratch_types=[
            pltpu.VMEM((_R, D), jnp.float32),
            pltpu.VMEM((_R, D), jnp.float32),
            pltpu.VMEM((_R, D), jnp.float32),
            pltpu.VMEM((_R, D), jnp.float32),
            pltpu.SemaphoreType.DMA,
            pltpu.SemaphoreType.DMA,
            pltpu.SemaphoreType.DMA,
            pltpu.SemaphoreType.DMA,
            pltpu.SemaphoreType.DMA,
            pltpu.SemaphoreType.DMA,
        ],
    )
    def sc_add(tok_hbm, pos_hbm, out_hbm, pos_a, pos_b, tok_a, tok_b,
               psem_a, psem_b, isem_a, isem_b, osem_a, osem_b):
        wid = lax.axis_index("s") * NC + lax.axis_index("c")
        s0 = wid * rows_per_w
        pos_bufs, psems = (pos_a, pos_b), (psem_a, psem_b)
        tok_bufs, isems = (tok_a, tok_b), (isem_a, isem_b)
        osems = (osem_a, osem_b)

        def pos_row(c):
            return s0 + c * _R

        def tok_row(k):
            c, b = divmod(k, B)
            return b * S + pos_row(c)

        pos_desc = [None, None]
        tok_desc = [None, None]
        out_desc = [None, None]
        pos_desc[0] = pltpu.async_copy(
            pos_hbm.at[pl.ds(pos_row(0), _R)], pos_bufs[0], psems[0])
        tok_desc[0] = pltpu.async_copy(
            tok_hbm.at[pl.ds(tok_row(0), _R)], tok_bufs[0], isems[0])

        for k in range(NIT):
            c, b = divmod(k, B)
            bi = k % 2
            pi = c % 2
            if b == 0:
                pos_desc[pi].wait()
                if c + 1 < CH:
                    ni = (c + 1) % 2
                    pos_desc[ni] = pltpu.async_copy(
                        pos_hbm.at[pl.ds(pos_row(c + 1), _R)], pos_bufs[ni], psems[ni])
            tok_desc[bi].wait()
            if k + 1 < NIT:
                ni = (k + 1) % 2
                if out_desc[ni] is not None:
                    out_desc[ni].wait()
                tok_desc[ni] = pltpu.async_copy(
                    tok_hbm.at[pl.ds(tok_row(k + 1), _R)], tok_bufs[ni], isems[ni])

            tok_v, pos_v = tok_bufs[bi], pos_bufs[pi]

            @plsc.parallel_loop(0, _R, 1)
            def add_row(r):
                @plsc.parallel_loop(0, D, step=16, unroll=8)
                def add16(j):
                    sl = pl.ds(j, 16)
                    plsc.addupdate(tok_v.at[r, sl], pos_v[r, sl])

            out_desc[bi] = pltpu.async_copy(
                tok_v, out_hbm.at[pl.ds(tok_row(k), _R)], osems[bi])

        for d in out_desc:
            if d is not None:
                d.wait()

    return sc_add


def kernel(encoded_tokens, pos_table):
    B, S, D = encoded_tokens.shape
    sc_add = _make_sc_add(B, S, D)
    out = sc_add(encoded_tokens.reshape(B * S, D), pos_table)
    return out.reshape(B, S, D)


# speedup vs baseline: 4.7679x; 1.0209x over previous
import functools

import jax
import jax.numpy as jnp
from jax import lax
from jax.experimental import pallas as pl
from jax.experimental.pallas import tpu as pltpu
from jax.experimental.pallas import tpu_sc as plsc

_R = 32


def _make_sc_add(B, S, D):
    info = plsc.get_sparse_core_info()
    NC, NS = info.num_cores, info.num_subcores
    NW = NC * NS
    rows_per_w = S // NW
    CH = rows_per_w // _R
    NIT = CH * B

    mesh = plsc.VectorSubcoreMesh(core_axis_name="c", subcore_axis_name="s")

    @functools.partial(
        pl.kernel,
        mesh=mesh,
        out_type=jax.ShapeDtypeStruct((B * S, D), jnp.float32),
        scratch_types=[
            pltpu.VMEM((_R, D), jnp.float32),
            pltpu.VMEM((_R, D), jnp.float32),
            pltpu.VMEM((_R, D), jnp.float32),
            pltpu.VMEM((_R, D), jnp.float32),
            pltpu.SemaphoreType.DMA,
            pltpu.SemaphoreType.DMA,
            pltpu.SemaphoreType.DMA,
            pltpu.SemaphoreType.DMA,
            pltpu.SemaphoreType.DMA,
            pltpu.SemaphoreType.DMA,
        ],
    )
    def sc_add(tok_hbm, pos_hbm, out_hbm, pos_a, pos_b, tok_a, tok_b,
               psem_a, psem_b, isem_a, isem_b, osem_a, osem_b):
        wid = lax.axis_index("s") * NC + lax.axis_index("c")
        s0 = wid * rows_per_w
        pos_bufs, psems = (pos_a, pos_b), (psem_a, psem_b)
        tok_bufs, isems = (tok_a, tok_b), (isem_a, isem_b)
        osems = (osem_a, osem_b)

        def pos_row(c):
            return s0 + c * _R

        def tok_row(k):
            c, b = divmod(k, B)
            return b * S + pos_row(c)

        pos_desc = [None, None]
        tok_desc = [None, None]
        out_desc = [None, None]
        pos_desc[0] = pltpu.async_copy(
            pos_hbm.at[pl.ds(pos_row(0), _R)], pos_bufs[0], psems[0])
        tok_desc[0] = pltpu.async_copy(
            tok_hbm.at[pl.ds(tok_row(0), _R)], tok_bufs[0], isems[0])

        for k in range(NIT):
            c, b = divmod(k, B)
            bi = k % 2
            pi = c % 2
            if b == 0:
                pos_desc[pi].wait()
                if c + 1 < CH:
                    ni = (c + 1) % 2
                    pos_desc[ni] = pltpu.async_copy(
                        pos_hbm.at[pl.ds(pos_row(c + 1), _R)], pos_bufs[ni], psems[ni])
            tok_desc[bi].wait()
            if k + 1 < NIT:
                ni = (k + 1) % 2
                if out_desc[ni] is not None:
                    out_desc[ni].wait()
                tok_desc[ni] = pltpu.async_copy(
                    tok_hbm.at[pl.ds(tok_row(k + 1), _R)], tok_bufs[ni], isems[ni])

            tok_v, pos_v = tok_bufs[bi], pos_bufs[pi]

            @plsc.parallel_loop(0, _R, 2)
            def add_row(r):
                @plsc.parallel_loop(0, D, step=16, unroll=8)
                def add16(j):
                    sl = pl.ds(j, 16)
                    plsc.addupdate(tok_v.at[r, sl], pos_v[r, sl])
                    plsc.addupdate(tok_v.at[r + 1, sl], pos_v[r + 1, sl])

            out_desc[bi] = pltpu.async_copy(
                tok_v, out_hbm.at[pl.ds(tok_row(k), _R)], osems[bi])

        for d in out_desc:
            if d is not None:
                d.wait()

    return sc_add


def kernel(encoded_tokens, pos_table):
    B, S, D = encoded_tokens.shape
    sc_add = _make_sc_add(B, S, D)
    out = sc_add(encoded_tokens.reshape(B * S, D), pos_table)
    return out.reshape(B, S, D)


# speedup vs baseline: 4.7888x; 1.0044x over previous
import functools

import jax
import jax.numpy as jnp
from jax import lax
from jax.experimental import pallas as pl
from jax.experimental.pallas import tpu as pltpu
from jax.experimental.pallas import tpu_sc as plsc

_R = 32


def _make_sc_add(B, S, D):
    info = plsc.get_sparse_core_info()
    NC, NS = info.num_cores, info.num_subcores
    NW = NC * NS
    rows_per_w = S // NW
    CH = rows_per_w // _R
    NIT = CH * B

    mesh = plsc.VectorSubcoreMesh(core_axis_name="c", subcore_axis_name="s")

    @functools.partial(
        pl.kernel,
        mesh=mesh,
        out_type=jax.ShapeDtypeStruct((B * S, D), jnp.float32),
        scratch_types=[
            pltpu.VMEM((_R, D), jnp.float32),
            pltpu.VMEM((_R, D), jnp.float32),
            pltpu.VMEM((_R, D), jnp.float32),
            pltpu.VMEM((_R, D), jnp.float32),
            pltpu.SemaphoreType.DMA,
            pltpu.SemaphoreType.DMA,
            pltpu.SemaphoreType.DMA,
            pltpu.SemaphoreType.DMA,
            pltpu.SemaphoreType.DMA,
            pltpu.SemaphoreType.DMA,
        ],
    )
    def sc_add(tok_hbm, pos_hbm, out_hbm, pos_a, pos_b, tok_a, tok_b,
               psem_a, psem_b, isem_a, isem_b, osem_a, osem_b):
        wid = lax.axis_index("s") * NC + lax.axis_index("c")
        s0 = wid * rows_per_w
        pos_bufs, psems = (pos_a, pos_b), (psem_a, psem_b)
        tok_bufs, isems = (tok_a, tok_b), (isem_a, isem_b)
        osems = (osem_a, osem_b)

        def pos_row(c):
            return s0 + c * _R

        def tok_row(k):
            c, b = divmod(k, B)
            return b * S + pos_row(c)

        pos_desc = [None, None]
        tok_desc = [None, None]
        out_desc = [None, None]
        pos_desc[0] = pltpu.async_copy(
            pos_hbm.at[pl.ds(pos_row(0), _R)], pos_bufs[0], psems[0])
        tok_desc[0] = pltpu.async_copy(
            tok_hbm.at[pl.ds(tok_row(0), _R)], tok_bufs[0], isems[0])

        for k in range(NIT):
            c, b = divmod(k, B)
            bi = k % 2
            pi = c % 2
            if b == 0:
                pos_desc[pi].wait()
                if c + 1 < CH:
                    ni = (c + 1) % 2
                    pos_desc[ni] = pltpu.async_copy(
                        pos_hbm.at[pl.ds(pos_row(c + 1), _R)], pos_bufs[ni], psems[ni])
            tok_desc[bi].wait()
            if k + 1 < NIT:
                ni = (k + 1) % 2
                if out_desc[ni] is not None:
                    out_desc[ni].wait()
                tok_desc[ni] = pltpu.async_copy(
                    tok_hbm.at[pl.ds(tok_row(k + 1), _R)], tok_bufs[ni], isems[ni])

            tok_v, pos_v = tok_bufs[bi], pos_bufs[pi]

            @plsc.parallel_loop(0, _R, 4)
            def add_row(r):
                @plsc.parallel_loop(0, D, step=16, unroll=4)
                def add16(j):
                    sl = pl.ds(j, 16)
                    for rr in range(4):
                        plsc.addupdate(tok_v.at[r + rr, sl], pos_v[r + rr, sl])

            out_desc[bi] = pltpu.async_copy(
                tok_v, out_hbm.at[pl.ds(tok_row(k), _R)], osems[bi])

        for d in out_desc:
            if d is not None:
                d.wait()

    return sc_add


def kernel(encoded_tokens, pos_table):
    B, S, D = encoded_tokens.shape
    sc_add = _make_sc_add(B, S, D)
    out = sc_add(encoded_tokens.reshape(B * S, D), pos_table)
    return out.reshape(B, S, D)
